# Initial kernel scaffold; baseline (speedup 1.0000x reference)
import jax
import jax.numpy as jnp
from jax.experimental import pallas as pl


def kernel(x, A, W, temperature):
    raise NotImplementedError("write your pallas kernel here")


# speedup vs baseline: 5.4228x; 5.4228x over previous
import functools

import jax
import jax.numpy as jnp
from jax import lax
from jax.experimental import pallas as pl
from jax.experimental.pallas import tpu as pltpu
from jax.experimental.pallas import tpu_sc as plsc

_N = 1024
_D = 128
_K = 8

_BR_EMBED = 256
_BR_KNN = 128

_NC = 2
_NS = 16
_NW = _NC * _NS
_CHUNK = (_N * _K) // _NW


def _rowsum(a):
    acc = a[:, 0:128]
    for c in range(1, 8):
        acc = acc + a[:, 128 * c:128 * (c + 1)]
    part = acc[:, 0:8]
    for v in range(1, 16):
        part = part + acc[:, 8 * v:8 * (v + 1)]
    q = part[:, 0:4] + part[:, 4:8]
    q = q[:, 0:2] + q[:, 2:4]
    return q[:, 0:1] + q[:, 1:2]


def _embed_body(a_ref, x_ref, w_ref, out_ref):
    a = a_ref[...]
    r = pl.reciprocal(_rowsum(a) + 1e-6, approx=False)
    t1 = jnp.dot(a * r, x_ref[...], preferred_element_type=jnp.float32)
    out_ref[...] = jnp.dot(t1, w_ref[...], preferred_element_type=jnp.float32)


def _embed(A, x, W):
    return pl.pallas_call(
        _embed_body,
        grid=(_N // _BR_EMBED,),
        in_specs=[
            pl.BlockSpec((_BR_EMBED, _N), lambda i: (i, 0)),
            pl.BlockSpec((_N, _D), lambda i: (0, 0)),
            pl.BlockSpec((_D, _D), lambda i: (0, 0)),
        ],
        out_specs=pl.BlockSpec((_BR_EMBED, _D), lambda i: (i, 0)),
        out_shape=jax.ShapeDtypeStruct((_N, _D), jnp.float32),
    )(A, x, W)


def _knn_body(t_ref, xeb_ref, xe_ref, lq_ref, idx_ref):
    j = pl.program_id(0)
    xeb = xeb_ref[...]
    xe = xe_ref[...]
    t = t_ref[0, 0]
    dots = lax.dot_general(
        xe, xeb, (((1,), (1,)), ((), ())), preferred_element_type=jnp.float32,
        precision=lax.Precision.HIGHEST)
    n2f = jnp.sum(xe * xe, axis=1, keepdims=True)
    ones = jnp.ones((1, _D), jnp.float32)
    n2b = lax.dot_general(
        ones, xeb * xeb, (((1,), (1,)), ((), ())),
        preferred_element_type=jnp.float32,
        precision=lax.Precision.HIGHEST)
    md = n2f + n2b - 2.0 * dots
    rows = lax.broadcasted_iota(jnp.int32, (_N, _BR_KNN), 0)
    cols = lax.broadcasted_iota(jnp.int32, (_N, _BR_KNN), 1) + j * _BR_KNN
    lq = jnp.where(rows == cols, 0.0, md * t)
    lq_ref[...] = lq

    work = lq
    big = jnp.float32(jnp.inf)
    bigi = jnp.int32(2**30)
    mins = []
    for _ in range(_K):
        mv = jnp.min(work, axis=0, keepdims=True)
        cand = jnp.where(work == mv, rows, bigi)
        am = jnp.min(cand, axis=0, keepdims=True)
        mins.append(am)
        work = jnp.where(rows == am, big, work)
    idx_ref[...] = jnp.concatenate(mins, axis=0)


def _knn(t2d, xe):
    return pl.pallas_call(
        _knn_body,
        grid=(_N // _BR_KNN,),
        in_specs=[
            pl.BlockSpec((1, 1), lambda j: (0, 0)),
            pl.BlockSpec((_BR_KNN, _D), lambda j: (j, 0)),
            pl.BlockSpec((_N, _D), lambda j: (0, 0)),
        ],
        out_specs=[
            pl.BlockSpec((_N, _BR_KNN), lambda j: (0, j)),
            pl.BlockSpec((_K, _BR_KNN), lambda j: (0, j)),
        ],
        out_shape=[
            jax.ShapeDtypeStruct((_N, _N), jnp.float32),
            jax.ShapeDtypeStruct((_K, _N), jnp.int32),
        ],
    )(t2d, xe, xe)


def _sc_gather_body(lq_hbm, idx_hbm, lp_out, src_out, rv, ga, gb, vv, sem):
    wid = lax.axis_index("s") * _NC + lax.axis_index("c")
    base = wid * _CHUNK
    pltpu.sync_copy(idx_hbm.at[pl.ds(base, _CHUNK)], rv)
    for i in range(_NS):
        u = lax.iota(jnp.int32, 16) + (i * 16)
        r = rv[pl.ds(i * 16, 16)]
        g = r * _N + jnp.right_shift(u + base, 3)
        if i < 8:
            ga[pl.ds(i * 16, 16)] = g
        else:
            gb[pl.ds((i - 8) * 16, 16)] = g
    pltpu.async_copy(lq_hbm.at[ga], vv.at[pl.ds(0, 128)], sem).wait()
    pltpu.async_copy(lq_hbm.at[gb], vv.at[pl.ds(128, 128)], sem).wait()
    for i in range(_NS):
        vv[pl.ds(i * 16, 16)] = -vv[pl.ds(i * 16, 16)]
    pltpu.sync_copy(vv, lp_out.at[pl.ds(base, _CHUNK)])
    pltpu.sync_copy(rv, src_out.at[pl.ds(base, _CHUNK)])


@functools.cache
def _sc_gather():
    return pl.kernel(
        _sc_gather_body,
        mesh=plsc.VectorSubcoreMesh(core_axis_name="c", subcore_axis_name="s"),
        out_type=[
            jax.ShapeDtypeStruct((_N * _K,), jnp.float32),
            jax.ShapeDtypeStruct((_N * _K,), jnp.int32),
        ],
        scratch_types=[
            pltpu.VMEM((_CHUNK,), jnp.int32),
            pltpu.VMEM((128,), jnp.int32),
            pltpu.VMEM((128,), jnp.int32),
            pltpu.VMEM((_CHUNK,), jnp.float32),
            pltpu.SemaphoreType.DMA,
        ],
    )


def kernel(x, A, W, temperature):
    t = jnp.exp(jnp.clip(temperature.astype(jnp.float32), -5.0, 5.0))
    xe = _embed(A, x, W)
    lq, idx = _knn(t.reshape(1, 1), xe)
    lp_flat, src = _sc_gather()(lq.reshape(-1), idx.reshape(-1))
    dst = jnp.arange(_N * _K, dtype=jnp.int32) // _K
    edges_hat = jnp.stack([src, dst], axis=-1).reshape(2, _N * _K)
    logprobs = lp_flat.reshape(1, _N, _K)
    return xe[None], edges_hat, logprobs


# speedup vs baseline: 5.5019x; 1.0146x over previous
import functools

import jax
import jax.numpy as jnp
from jax import lax
from jax.experimental import pallas as pl
from jax.experimental.pallas import tpu as pltpu
from jax.experimental.pallas import tpu_sc as plsc

_N = 1024
_D = 128
_K = 8

_BR_EMBED = 256
_BR_KNN = 128

_NC = 2
_NS = 16
_NW = _NC * _NS
_CHUNK = (_N * _K) // _NW


def _rowsum(a):
    acc = a[:, 0:128]
    for c in range(1, 8):
        acc = acc + a[:, 128 * c:128 * (c + 1)]
    part = acc[:, 0:8]
    for v in range(1, 16):
        part = part + acc[:, 8 * v:8 * (v + 1)]
    q = part[:, 0:4] + part[:, 4:8]
    q = q[:, 0:2] + q[:, 2:4]
    return q[:, 0:1] + q[:, 1:2]


def _embed_body(a_ref, x_ref, w_ref, out_ref):
    a = a_ref[...]
    r = pl.reciprocal(_rowsum(a) + 1e-6, approx=False)
    t1 = jnp.dot(a * r, x_ref[...], preferred_element_type=jnp.float32)
    out_ref[...] = jnp.dot(t1, w_ref[...], preferred_element_type=jnp.float32)


def _embed(A, x, W):
    return pl.pallas_call(
        _embed_body,
        grid=(_N // _BR_EMBED,),
        in_specs=[
            pl.BlockSpec((_BR_EMBED, _N), lambda i: (i, 0)),
            pl.BlockSpec((_N, _D), lambda i: (0, 0)),
            pl.BlockSpec((_D, _D), lambda i: (0, 0)),
        ],
        out_specs=pl.BlockSpec((_BR_EMBED, _D), lambda i: (i, 0)),
        out_shape=jax.ShapeDtypeStruct((_N, _D), jnp.float32),
    )(A, x, W)


def _knn_body(t_ref, xeb_ref, xe_ref, lq_ref, idx_ref):
    j = pl.program_id(0)
    xeb = xeb_ref[...]
    xe = xe_ref[...]
    t = t_ref[0, 0]
    dots = lax.dot_general(
        xe, xeb, (((1,), (1,)), ((), ())), preferred_element_type=jnp.float32,
        precision=lax.Precision.HIGHEST)
    n2f = jnp.sum(xe * xe, axis=1, keepdims=True)
    ones = jnp.ones((1, _D), jnp.float32)
    n2b = lax.dot_general(
        ones, xeb * xeb, (((1,), (1,)), ((), ())),
        preferred_element_type=jnp.float32,
        precision=lax.Precision.HIGHEST)
    md = n2f + n2b - 2.0 * dots
    rows = lax.broadcasted_iota(jnp.int32, (_N, _BR_KNN), 0)
    cols = lax.broadcasted_iota(jnp.int32, (_N, _BR_KNN), 1) + j * _BR_KNN
    scaled = md * t
    lq_ref[...] = jnp.where(rows == cols, 0.0, scaled)

    big = jnp.float32(jnp.inf)
    bigi = jnp.int32(2**30)
    work = jnp.where(rows == cols, big, scaled)
    mins = [cols[0:1, :]]
    for _ in range(_K - 1):
        mv = jnp.min(work, axis=0, keepdims=True)
        cand = jnp.where(work == mv, rows, bigi)
        am = jnp.min(cand, axis=0, keepdims=True)
        mins.append(am)
        work = jnp.where(rows == am, big, work)
    idx_ref[...] = jnp.concatenate(mins, axis=0)


def _knn(t2d, xe):
    return pl.pallas_call(
        _knn_body,
        grid=(_N // _BR_KNN,),
        in_specs=[
            pl.BlockSpec((1, 1), lambda j: (0, 0)),
            pl.BlockSpec((_BR_KNN, _D), lambda j: (j, 0)),
            pl.BlockSpec((_N, _D), lambda j: (0, 0)),
        ],
        out_specs=[
            pl.BlockSpec((_N, _BR_KNN), lambda j: (0, j)),
            pl.BlockSpec((_K, _BR_KNN), lambda j: (0, j)),
        ],
        out_shape=[
            jax.ShapeDtypeStruct((_N, _N), jnp.float32),
            jax.ShapeDtypeStruct((_K, _N), jnp.int32),
        ],
    )(t2d, xe, xe)


def _sc_gather_body(lq_hbm, idx_hbm, lp_out, edge_out,
                    rv, ga, gb, dup, vv, ev, sem):
    wid = lax.axis_index("s") * _NC + lax.axis_index("c")
    base = wid * _CHUNK
    pltpu.sync_copy(idx_hbm.at[pl.ds(base, _CHUNK)], rv)
    lane = lax.iota(jnp.int32, 16)
    for i in range(_NS):
        u = lane + (i * 16)
        r = rv[pl.ds(i * 16, 16)]
        g = r * _N + jnp.right_shift(u + base, 3)
        if i < 8:
            ga[pl.ds(i * 16, 16)] = g
        else:
            gb[pl.ds((i - 8) * 16, 16)] = g
    for j in range(2 * _NS):
        dup[pl.ds(j * 16, 16)] = base + 8 * j + jnp.right_shift(lane, 1)
    pltpu.async_copy(lq_hbm.at[ga], vv.at[pl.ds(0, 128)], sem).wait()
    pltpu.async_copy(lq_hbm.at[gb], vv.at[pl.ds(128, 128)], sem).wait()
    for h in range(4):
        pltpu.async_copy(idx_hbm.at[dup.at[pl.ds(h * 128, 128)]],
                         ev.at[pl.ds(h * 128, 128)], sem).wait()
    even = (lane & 1) == 0
    for i in range(_NS):
        vv[pl.ds(i * 16, 16)] = -vv[pl.ds(i * 16, 16)]
    for j in range(2 * _NS):
        m = dup[pl.ds(j * 16, 16)]
        ev[pl.ds(j * 16, 16)] = jnp.where(
            even, ev[pl.ds(j * 16, 16)], jnp.right_shift(m, 3))
    pltpu.sync_copy(vv, lp_out.at[pl.ds(base, _CHUNK)])
    pltpu.sync_copy(ev, edge_out.at[pl.ds(base * 2, _CHUNK * 2)])


@functools.cache
def _sc_gather():
    return pl.kernel(
        _sc_gather_body,
        mesh=plsc.VectorSubcoreMesh(core_axis_name="c", subcore_axis_name="s"),
        out_type=[
            jax.ShapeDtypeStruct((_N * _K,), jnp.float32),
            jax.ShapeDtypeStruct((_N * _K * 2,), jnp.int32),
        ],
        scratch_types=[
            pltpu.VMEM((_CHUNK,), jnp.int32),
            pltpu.VMEM((128,), jnp.int32),
            pltpu.VMEM((128,), jnp.int32),
            pltpu.VMEM((_CHUNK * 2,), jnp.int32),
            pltpu.VMEM((_CHUNK,), jnp.float32),
            pltpu.VMEM((_CHUNK * 2,), jnp.int32),
            pltpu.SemaphoreType.DMA,
        ],
    )


def kernel(x, A, W, temperature):
    t = jnp.exp(jnp.clip(temperature.astype(jnp.float32), -5.0, 5.0))
    xe = _embed(A, x, W)
    lq, idx = _knn(t.reshape(1, 1), xe)
    lp_flat, edge_flat = _sc_gather()(lq.reshape(-1), idx.reshape(-1))
    edges_hat = edge_flat.reshape(2, _N * _K)
    logprobs = lp_flat.reshape(1, _N, _K)
    return xe[None], edges_hat, logprobs


# speedup vs baseline: 6.3160x; 1.1480x over previous
import functools

import jax
import jax.numpy as jnp
from jax import lax
from jax.experimental import pallas as pl
from jax.experimental.pallas import tpu as pltpu
from jax.experimental.pallas import tpu_sc as plsc

_N = 1024
_D = 128
_K = 8

_BR_EMBED = 256
_BR_KNN = 256

_NC = 2
_NS = 16
_NW = _NC * _NS
_CHUNK = (_N * _K) // _NW


def _rowsum(a):
    acc = a[:, 0:128]
    for c in range(1, 8):
        acc = acc + a[:, 128 * c:128 * (c + 1)]
    part = acc[:, 0:8]
    for v in range(1, 16):
        part = part + acc[:, 8 * v:8 * (v + 1)]
    q = part[:, 0:4] + part[:, 4:8]
    q = q[:, 0:2] + q[:, 2:4]
    return q[:, 0:1] + q[:, 1:2]


def _embed_body(a_ref, x_ref, w_ref, out_ref):
    a = a_ref[...]
    r = pl.reciprocal(_rowsum(a) + 1e-6, approx=False)
    t1 = jnp.dot(a * r, x_ref[...], preferred_element_type=jnp.float32)
    out_ref[...] = jnp.dot(t1, w_ref[...], preferred_element_type=jnp.float32)


def _embed(A, x, W):
    return pl.pallas_call(
        _embed_body,
        grid=(_N // _BR_EMBED,),
        in_specs=[
            pl.BlockSpec((_BR_EMBED, _N), lambda i: (i, 0)),
            pl.BlockSpec((_N, _D), lambda i: (0, 0)),
            pl.BlockSpec((_D, _D), lambda i: (0, 0)),
        ],
        out_specs=pl.BlockSpec((_BR_EMBED, _D), lambda i: (i, 0)),
        out_shape=jax.ShapeDtypeStruct((_N, _D), jnp.float32),
    )(A, x, W)


def _knn_body(t_ref, xeb_ref, xe_ref, lq_ref, idx_ref):
    j = pl.program_id(0)
    xeb = xeb_ref[...]
    xe = xe_ref[...]
    t = t_ref[0, 0]
    dots = lax.dot_general(
        xe, xeb, (((1,), (1,)), ((), ())), preferred_element_type=jnp.float32,
        precision=lax.Precision.HIGHEST)
    n2f = jnp.sum(xe * xe, axis=1, keepdims=True)
    ones = jnp.ones((1, _D), jnp.float32)
    n2b = lax.dot_general(
        ones, xeb * xeb, (((1,), (1,)), ((), ())),
        preferred_element_type=jnp.float32,
        precision=lax.Precision.HIGHEST)
    md = n2f + n2b - 2.0 * dots
    rows = lax.broadcasted_iota(jnp.int32, (_N, _BR_KNN), 0)
    cols = lax.broadcasted_iota(jnp.int32, (_N, _BR_KNN), 1) + j * _BR_KNN
    scaled = md * t
    lq_ref[...] = jnp.where(rows == cols, 0.0, scaled)

    big = jnp.float32(jnp.inf)
    bigi = jnp.int32(2**30)
    work = jnp.where(rows == cols, big, scaled)
    mins = [cols[0:1, :]]
    for _ in range(_K - 1):
        mv = jnp.min(work, axis=0, keepdims=True)
        cand = jnp.where(work == mv, rows, bigi)
        am = jnp.min(cand, axis=0, keepdims=True)
        mins.append(am)
        work = jnp.where(rows == am, big, work)
    idx_ref[...] = jnp.concatenate(mins, axis=0)


def _knn(t2d, xe):
    return pl.pallas_call(
        _knn_body,
        grid=(_N // _BR_KNN,),
        in_specs=[
            pl.BlockSpec((1, 1), lambda j: (0, 0)),
            pl.BlockSpec((_BR_KNN, _D), lambda j: (j, 0)),
            pl.BlockSpec((_N, _D), lambda j: (0, 0)),
        ],
        out_specs=[
            pl.BlockSpec((_N, _BR_KNN), lambda j: (0, j)),
            pl.BlockSpec((_K, _BR_KNN), lambda j: (0, j)),
        ],
        out_shape=[
            jax.ShapeDtypeStruct((_N, _N), jnp.float32),
            jax.ShapeDtypeStruct((_K, _N), jnp.int32),
        ],
    )(t2d, xe, xe)


def _sc_gather_body(lq_hbm, idx_hbm, lp_out, edge_out,
                    rv, ga, gb, dup, vv, ev, sem):
    wid = lax.axis_index("s") * _NC + lax.axis_index("c")
    base = wid * _CHUNK
    pltpu.sync_copy(idx_hbm.at[pl.ds(base, _CHUNK)], rv)
    lane = lax.iota(jnp.int32, 16)
    for i in range(_NS):
        u = lane + (i * 16)
        r = rv[pl.ds(i * 16, 16)]
        g = r * _N + jnp.right_shift(u + base, 3)
        if i < 8:
            ga[pl.ds(i * 16, 16)] = g
        else:
            gb[pl.ds((i - 8) * 16, 16)] = g
    for j in range(2 * _NS):
        dup[pl.ds(j * 16, 16)] = base + 8 * j + jnp.right_shift(lane, 1)
    copies = [
        pltpu.async_copy(lq_hbm.at[ga], vv.at[pl.ds(0, 128)], sem),
        pltpu.async_copy(lq_hbm.at[gb], vv.at[pl.ds(128, 128)], sem),
    ]
    for h in range(4):
        copies.append(
            pltpu.async_copy(idx_hbm.at[dup.at[pl.ds(h * 128, 128)]],
                             ev.at[pl.ds(h * 128, 128)], sem))
    for c in copies:
        c.wait()
    even = (lane & 1) == 0
    for i in range(_NS):
        vv[pl.ds(i * 16, 16)] = -vv[pl.ds(i * 16, 16)]
    for j in range(2 * _NS):
        m = dup[pl.ds(j * 16, 16)]
        ev[pl.ds(j * 16, 16)] = jnp.where(
            even, ev[pl.ds(j * 16, 16)], jnp.right_shift(m, 3))
    pltpu.sync_copy(vv, lp_out.at[pl.ds(base, _CHUNK)])
    pltpu.sync_copy(ev, edge_out.at[pl.ds(base * 2, _CHUNK * 2)])


@functools.cache
def _sc_gather():
    return pl.kernel(
        _sc_gather_body,
        mesh=plsc.VectorSubcoreMesh(core_axis_name="c", subcore_axis_name="s"),
        out_type=[
            jax.ShapeDtypeStruct((_N * _K,), jnp.float32),
            jax.ShapeDtypeStruct((_N * _K * 2,), jnp.int32),
        ],
        scratch_types=[
            pltpu.VMEM((_CHUNK,), jnp.int32),
            pltpu.VMEM((128,), jnp.int32),
            pltpu.VMEM((128,), jnp.int32),
            pltpu.VMEM((_CHUNK * 2,), jnp.int32),
            pltpu.VMEM((_CHUNK,), jnp.float32),
            pltpu.VMEM((_CHUNK * 2,), jnp.int32),
            pltpu.SemaphoreType.DMA,
        ],
    )


def kernel(x, A, W, temperature):
    t = jnp.exp(jnp.clip(temperature.astype(jnp.float32), -5.0, 5.0))
    xe = _embed(A, x, W)
    lq, idx = _knn(t.reshape(1, 1), xe)
    lp_flat, edge_flat = _sc_gather()(lq.reshape(-1), idx.reshape(-1))
    edges_hat = edge_flat.reshape(2, _N * _K)
    logprobs = lp_flat.reshape(1, _N, _K)
    return xe[None], edges_hat, logprobs


# speedup vs baseline: 6.5636x; 1.0392x over previous
import functools

import jax
import jax.numpy as jnp
from jax import lax
from jax.experimental import pallas as pl
from jax.experimental.pallas import tpu as pltpu
from jax.experimental.pallas import tpu_sc as plsc

_N = 1024
_D = 128
_K = 8

_BR_EMBED = 256
_BR_KNN = 512

_NC = 2
_NS = 16
_NW = _NC * _NS
_CHUNK = (_N * _K) // _NW


def _rowsum(a):
    acc = a[:, 0:128]
    for c in range(1, 8):
        acc = acc + a[:, 128 * c:128 * (c + 1)]
    part = acc[:, 0:8]
    for v in range(1, 16):
        part = part + acc[:, 8 * v:8 * (v + 1)]
    q = part[:, 0:4] + part[:, 4:8]
    q = q[:, 0:2] + q[:, 2:4]
    return q[:, 0:1] + q[:, 1:2]


def _embed_body(a_ref, x_ref, w_ref, out_ref):
    a = a_ref[...]
    r = pl.reciprocal(_rowsum(a) + 1e-6, approx=False)
    t1 = jnp.dot(a * r, x_ref[...], preferred_element_type=jnp.float32)
    out_ref[...] = jnp.dot(t1, w_ref[...], preferred_element_type=jnp.float32)


def _embed(A, x, W):
    return pl.pallas_call(
        _embed_body,
        grid=(_N // _BR_EMBED,),
        in_specs=[
            pl.BlockSpec((_BR_EMBED, _N), lambda i: (i, 0)),
            pl.BlockSpec((_N, _D), lambda i: (0, 0)),
            pl.BlockSpec((_D, _D), lambda i: (0, 0)),
        ],
        out_specs=pl.BlockSpec((_BR_EMBED, _D), lambda i: (i, 0)),
        out_shape=jax.ShapeDtypeStruct((_N, _D), jnp.float32),
    )(A, x, W)


def _knn_body(t_ref, xeb_ref, xe_ref, lq_ref, idx_ref):
    j = pl.program_id(0)
    xeb = xeb_ref[...]
    xe = xe_ref[...]
    t = t_ref[0, 0]
    dots = lax.dot_general(
        xe, xeb, (((1,), (1,)), ((), ())), preferred_element_type=jnp.float32,
        precision=lax.Precision.HIGHEST)
    n2f = jnp.sum(xe * xe, axis=1, keepdims=True)
    ones = jnp.ones((1, _D), jnp.float32)
    n2b = lax.dot_general(
        ones, xeb * xeb, (((1,), (1,)), ((), ())),
        preferred_element_type=jnp.float32,
        precision=lax.Precision.HIGHEST)
    md = n2f + n2b - 2.0 * dots
    rows = lax.broadcasted_iota(jnp.int32, (_N, _BR_KNN), 0)
    cols = lax.broadcasted_iota(jnp.int32, (_N, _BR_KNN), 1) + j * _BR_KNN
    scaled = md * t
    lq_ref[...] = jnp.where(rows == cols, 0.0, scaled)

    big = jnp.float32(jnp.inf)
    bigi = jnp.int32(2**30)
    work = jnp.where(rows == cols, big, scaled)
    mins = [cols[0:1, :]]
    for _ in range(_K - 1):
        mv = jnp.min(work, axis=0, keepdims=True)
        cand = jnp.where(work == mv, rows, bigi)
        am = jnp.min(cand, axis=0, keepdims=True)
        mins.append(am)
        work = jnp.where(rows == am, big, work)
    idx_ref[...] = jnp.concatenate(mins, axis=0)


def _knn(t2d, xe):
    return pl.pallas_call(
        _knn_body,
        grid=(_N // _BR_KNN,),
        in_specs=[
            pl.BlockSpec((1, 1), lambda j: (0, 0)),
            pl.BlockSpec((_BR_KNN, _D), lambda j: (j, 0)),
            pl.BlockSpec((_N, _D), lambda j: (0, 0)),
        ],
        out_specs=[
            pl.BlockSpec((_N, _BR_KNN), lambda j: (0, j)),
            pl.BlockSpec((_K, _BR_KNN), lambda j: (0, j)),
        ],
        out_shape=[
            jax.ShapeDtypeStruct((_N, _N), jnp.float32),
            jax.ShapeDtypeStruct((_K, _N), jnp.int32),
        ],
    )(t2d, xe, xe)


def _sc_gather_body(lq_hbm, idx_hbm, lp_out, edge_out,
                    rv, ga, gb, dup, vv, ev, sem):
    wid = lax.axis_index("s") * _NC + lax.axis_index("c")
    base = wid * _CHUNK
    pltpu.sync_copy(idx_hbm.at[pl.ds(base, _CHUNK)], rv)
    lane = lax.iota(jnp.int32, 16)
    for i in range(_NS):
        u = lane + (i * 16)
        r = rv[pl.ds(i * 16, 16)]
        g = r * _N + jnp.right_shift(u + base, 3)
        if i < 8:
            ga[pl.ds(i * 16, 16)] = g
        else:
            gb[pl.ds((i - 8) * 16, 16)] = g
    for j in range(2 * _NS):
        dup[pl.ds(j * 16, 16)] = base + 8 * j + jnp.right_shift(lane, 1)
    copies = [
        pltpu.async_copy(lq_hbm.at[ga], vv.at[pl.ds(0, 128)], sem),
        pltpu.async_copy(lq_hbm.at[gb], vv.at[pl.ds(128, 128)], sem),
    ]
    for h in range(4):
        copies.append(
            pltpu.async_copy(idx_hbm.at[dup.at[pl.ds(h * 128, 128)]],
                             ev.at[pl.ds(h * 128, 128)], sem))
    for c in copies:
        c.wait()
    even = (lane & 1) == 0
    for i in range(_NS):
        vv[pl.ds(i * 16, 16)] = -vv[pl.ds(i * 16, 16)]
    for j in range(2 * _NS):
        m = dup[pl.ds(j * 16, 16)]
        ev[pl.ds(j * 16, 16)] = jnp.where(
            even, ev[pl.ds(j * 16, 16)], jnp.right_shift(m, 3))
    pltpu.sync_copy(vv, lp_out.at[pl.ds(base, _CHUNK)])
    pltpu.sync_copy(ev, edge_out.at[pl.ds(base * 2, _CHUNK * 2)])


@functools.cache
def _sc_gather():
    return pl.kernel(
        _sc_gather_body,
        mesh=plsc.VectorSubcoreMesh(core_axis_name="c", subcore_axis_name="s"),
        out_type=[
            jax.ShapeDtypeStruct((_N * _K,), jnp.float32),
            jax.ShapeDtypeStruct((_N * _K * 2,), jnp.int32),
        ],
        scratch_types=[
            pltpu.VMEM((_CHUNK,), jnp.int32),
            pltpu.VMEM((128,), jnp.int32),
            pltpu.VMEM((128,), jnp.int32),
            pltpu.VMEM((_CHUNK * 2,), jnp.int32),
            pltpu.VMEM((_CHUNK,), jnp.float32),
            pltpu.VMEM((_CHUNK * 2,), jnp.int32),
            pltpu.SemaphoreType.DMA,
        ],
    )


def kernel(x, A, W, temperature):
    t = jnp.exp(jnp.clip(temperature.astype(jnp.float32), -5.0, 5.0))
    xe = _embed(A, x, W)
    lq, idx = _knn(t.reshape(1, 1), xe)
    lp_flat, edge_flat = _sc_gather()(lq.reshape(-1), idx.reshape(-1))
    edges_hat = edge_flat.reshape(2, _N * _K)
    logprobs = lp_flat.reshape(1, _N, _K)
    return xe[None], edges_hat, logprobs
